# baseline (speedup 1.0000x reference)
import functools

import jax
import jax.numpy as jnp
from jax import lax
from jax.experimental import pallas as pl
from jax.experimental.pallas import tpu as pltpu
from jax.experimental.pallas import tpu_sc as plsc

NC = 2
NS = 16
CHUNK = 512
NBUF = 2


def _sc_degree(dst_w, ones_in, zeros, npad):
  nw, nch, ch = dst_w.shape
  rpt = npad // NS
  mesh = plsc.VectorSubcoreMesh(core_axis_name="c", subcore_axis_name="s", num_cores=NC, num_subcores=NS)

  @functools.partial(
      pl.kernel,
      out_type=jax.ShapeDtypeStruct((NC, npad, 16), jnp.float32),
      mesh=mesh,
      scratch_types=[
          pltpu.VMEM((nch, ch), jnp.int32),
          pltpu.VMEM((ch, 16), jnp.float32),
          pltpu.VMEM_SHARED((npad, 16), jnp.float32),
      ],
      compiler_params=pltpu.CompilerParams(use_tc_tiling_on_sc=False),
  )
  def k(dst_hbm, ones_hbm, zero_hbm, out_hbm, idx_v, ones_v, acc_sh):
    cid = lax.axis_index("c")
    sid = lax.axis_index("s")
    wid = cid * NS + sid
    pltpu.sync_copy(zero_hbm.at[pl.ds(sid * rpt, rpt)],
                    acc_sh.at[pl.ds(sid * rpt, rpt)])
    pltpu.sync_copy(ones_hbm, ones_v)
    pltpu.sync_copy(dst_hbm.at[wid], idx_v)
    plsc.subcore_barrier()

    def body(c, carry):
      pltpu.sync_copy(ones_v, acc_sh.at[idx_v.at[c]], add=True)
      return carry

    lax.fori_loop(0, nch, body, 0)
    plsc.subcore_barrier()
    pltpu.sync_copy(acc_sh.at[pl.ds(sid * rpt, rpt)],
                    out_hbm.at[cid, pl.ds(sid * rpt, rpt)])

  return k(dst_w, ones_in, zeros)


def _sc_spmm(table, src_w, dst_w, zeros, npad):
  hsp, _, dh = table.shape
  nw, nch, ch = src_w.shape
  rpt = npad // NS
  mesh = plsc.VectorSubcoreMesh(core_axis_name="c", subcore_axis_name="s", num_cores=NC, num_subcores=NS)

  @functools.partial(
      pl.kernel,
      out_type=jax.ShapeDtypeStruct((NC, hsp, npad, dh), jnp.float32),
      mesh=mesh,
      scratch_types=[
          pltpu.VMEM((nch, ch), jnp.int32),
          pltpu.VMEM((nch, ch), jnp.int32),
          pltpu.VMEM((NBUF, ch, dh), jnp.float32),
          pltpu.VMEM_SHARED((npad, dh), jnp.float32),
          pltpu.SemaphoreType.DMA,
          pltpu.SemaphoreType.DMA,
      ],
      compiler_params=pltpu.CompilerParams(use_tc_tiling_on_sc=False),
  )
  def k(tab_hbm, src_hbm, dst_hbm, zero_hbm, out_hbm,
        sidx, didx, rows, acc_sh, sem0, sem1):
    sems = [sem0, sem1]
    cid = lax.axis_index("c")
    sid = lax.axis_index("s")
    wid = cid * NS + sid
    pltpu.sync_copy(src_hbm.at[wid], sidx)
    pltpu.sync_copy(dst_hbm.at[wid], didx)
    for h in range(hsp):
      tab_h = tab_hbm.at[h]
      pltpu.sync_copy(zero_hbm.at[pl.ds(sid * rpt, rpt)],
                      acc_sh.at[pl.ds(sid * rpt, rpt)])
      plsc.subcore_barrier()
      for b in range(NBUF):
        pltpu.make_async_copy(tab_h.at[sidx.at[b]], rows.at[b],
                              sems[b]).start()

      def outer(i, carry):
        c0 = i * NBUF
        for b in range(NBUF):
          c = c0 + b
          pltpu.make_async_copy(tab_h.at[sidx.at[c]], rows.at[b],
                                sems[b]).wait()
          pltpu.sync_copy(rows.at[b], acc_sh.at[didx.at[c]], add=True)
          nxt = c + NBUF

          @pl.when(nxt < nch)
          def _():
            pltpu.make_async_copy(tab_h.at[sidx.at[nxt]], rows.at[b],
                                  sems[b]).start()

        return carry

      lax.fori_loop(0, nch // NBUF, outer, 0)
      plsc.subcore_barrier()
      pltpu.sync_copy(acc_sh.at[pl.ds(sid * rpt, rpt)],
                      out_hbm.at[cid, h, pl.ds(sid * rpt, rpt)])

  return k(table, src_w, dst_w, zeros)


def _tc_matmul(x, w):
  n, kdim = x.shape
  m = w.shape[1]
  bm = 2000

  def body(x_ref, w_ref, o_ref):
    o_ref[...] = jnp.dot(x_ref[...], w_ref[...],
                         preferred_element_type=jnp.float32)

  return pl.pallas_call(
      body,
      grid=(n // bm,),
      in_specs=[pl.BlockSpec((bm, kdim), lambda i: (i, 0)),
                pl.BlockSpec((kdim, m), lambda i: (0, 0))],
      out_specs=pl.BlockSpec((bm, m), lambda i: (i, 0)),
      out_shape=jax.ShapeDtypeStruct((n, m), jnp.float32),
  )(x, w)


def _tc_scale(deg, h1):
  _, n, _ = deg.shape
  hid = h1.shape[1]
  bm = 2000

  hsp = 2
  dh = hid // hsp

  def body(deg_ref, h_ref, dinv_ref, ht_ref):
    degsum = deg_ref[0] + deg_ref[1] + 1.0
    dinv16 = lax.rsqrt(jnp.maximum(degsum, 1e-12))
    dinv_b = jnp.broadcast_to(dinv16[:, :1], (bm, hid))
    dinv_ref[...] = dinv_b
    ht = dinv_b * h_ref[...]
    for h in range(hsp):
      ht_ref[h] = ht[:, h * dh:(h + 1) * dh]

  return pl.pallas_call(
      body,
      grid=(n // bm,),
      in_specs=[pl.BlockSpec((2, bm, 16), lambda i: (0, i, 0)),
                pl.BlockSpec((bm, hid), lambda i: (i, 0))],
      out_specs=[pl.BlockSpec((bm, hid), lambda i: (i, 0)),
                 pl.BlockSpec((hsp, bm, dh), lambda i: (0, i, 0))],
      out_shape=[jax.ShapeDtypeStruct((n, hid), jnp.float32),
                 jax.ShapeDtypeStruct((hsp, n, dh), jnp.float32)],
  )(deg, h1)


def _tc_layer2(acc1, ht1, dinv_b, b1, w2):
  _, hsp, n, dh = acc1.shape
  hid = hsp * dh
  ncls = w2.shape[1]
  bm = 2000

  def body(acc_ref, ht_ref, dinv_ref, b1_ref, w2_ref, o_ref):
    accsum = acc_ref[0] + acc_ref[1]
    acc = jnp.concatenate([accsum[h] for h in range(hsp)], axis=-1)
    ht = jnp.concatenate([ht_ref[h] for h in range(hsp)], axis=-1)
    agg = dinv_ref[...] * (acc + ht) + b1_ref[...]
    z = jnp.maximum(agg, 0.0)
    h2 = jnp.dot(z, w2_ref[...], preferred_element_type=jnp.float32)
    o_ref[0] = h2 * dinv_ref[:, :ncls]

  return pl.pallas_call(
      body,
      grid=(n // bm,),
      in_specs=[pl.BlockSpec((2, hsp, bm, dh), lambda i: (0, 0, i, 0)),
                pl.BlockSpec((hsp, bm, dh), lambda i: (0, i, 0)),
                pl.BlockSpec((bm, hid), lambda i: (i, 0)),
                pl.BlockSpec((1, hid), lambda i: (0, 0)),
                pl.BlockSpec((hid, ncls), lambda i: (0, 0))],
      out_specs=pl.BlockSpec((1, bm, ncls), lambda i: (0, i, 0)),
      out_shape=jax.ShapeDtypeStruct((1, n, ncls), jnp.float32),
  )(acc1, ht1, dinv_b, b1, w2)


def _tc_final(acc2, ht2, dinv_b, b2):
  _, _, n, ncls = acc2.shape
  hid = dinv_b.shape[1]
  bm = 2000

  def body(acc_ref, ht_ref, dinv_ref, b2_ref, o_ref):
    o = dinv_ref[:, :ncls] * (acc_ref[0, 0] + acc_ref[1, 0] + ht_ref[0])
    o = o + b2_ref[...]
    m = jnp.max(o, axis=-1, keepdims=True)
    ex = jnp.exp(o - m)
    lse = jnp.log(jnp.sum(ex, axis=-1, keepdims=True)) + m
    o_ref[...] = o - lse

  return pl.pallas_call(
      body,
      grid=(n // bm,),
      in_specs=[pl.BlockSpec((2, 1, bm, ncls), lambda i: (0, 0, i, 0)),
                pl.BlockSpec((1, bm, ncls), lambda i: (0, i, 0)),
                pl.BlockSpec((bm, hid), lambda i: (i, 0)),
                pl.BlockSpec((1, ncls), lambda i: (0, 0))],
      out_specs=pl.BlockSpec((bm, ncls), lambda i: (i, 0)),
      out_shape=jax.ShapeDtypeStruct((n, ncls), jnp.float32),
  )(acc2, ht2, dinv_b, b2)


def kernel(x, edge_index, W1, b1, W2, b2):
  n, _ = x.shape
  hid = W1.shape[1]
  ncls = W2.shape[1]
  e = edge_index.shape[1]
  nw = NC * NS
  npad = ((n + 1 + 127) // 128) * 128
  nch = -(-e // (nw * CHUNK))
  nch = ((nch + NBUF - 1) // NBUF) * NBUF
  epad = nw * nch * CHUNK

  src = edge_index[0]
  dst = edge_index[1]
  pad = epad - e
  srcp = jnp.concatenate([src, jnp.zeros((pad,), jnp.int32)]).reshape(
      nw, nch, CHUNK)
  dstp = jnp.concatenate([dst, jnp.full((pad,), n, jnp.int32)]).reshape(
      nw, nch, CHUNK)
  zeros16 = jnp.zeros((npad, 16), jnp.float32)
  zeros_h = jnp.zeros((npad, hid // 2), jnp.float32)
  zeros_c = jnp.zeros((npad, ncls), jnp.float32)
  ones_in = jnp.ones((CHUNK, 16), jnp.float32)

  deg_parts = _sc_degree(dstp, ones_in, zeros16, npad)
  h1 = _tc_matmul(x, W1)
  dinv_b, ht1 = _tc_scale(deg_parts[:, :n], h1)
  acc1 = _sc_spmm(ht1, srcp, dstp, zeros_h, npad)
  ht2 = _tc_layer2(acc1[:, :, :n], ht1, dinv_b, b1.reshape(1, -1), W2)
  acc2 = _sc_spmm(ht2, srcp, dstp, zeros_c, npad)
  return _tc_final(acc2[:, :, :n], ht2, dinv_b, b2.reshape(1, -1))


# speedup vs baseline: 1.9081x; 1.9081x over previous
import functools

import jax
import jax.numpy as jnp
from jax import lax
from jax.experimental import pallas as pl
from jax.experimental.pallas import tpu as pltpu
from jax.experimental.pallas import tpu_sc as plsc

NC = 2
NS = 16
CHUNK = 512
NBUF = 2


def _sc_degree(dst_w, ones_in, zeros, npad):
  nw, nch, ch = dst_w.shape
  rpt = npad // NS
  mesh = plsc.VectorSubcoreMesh(core_axis_name="c", subcore_axis_name="s", num_cores=NC, num_subcores=NS)

  @functools.partial(
      pl.kernel,
      out_type=jax.ShapeDtypeStruct((NC, npad, 16), jnp.float32),
      mesh=mesh,
      scratch_types=[
          pltpu.VMEM((nch, ch), jnp.int32),
          pltpu.VMEM((ch, 16), jnp.float32),
          pltpu.VMEM_SHARED((npad, 16), jnp.float32),
      ],
      compiler_params=pltpu.CompilerParams(use_tc_tiling_on_sc=False),
  )
  def k(dst_hbm, ones_hbm, zero_hbm, out_hbm, idx_v, ones_v, acc_sh):
    cid = lax.axis_index("c")
    sid = lax.axis_index("s")
    wid = cid * NS + sid
    pltpu.sync_copy(zero_hbm.at[pl.ds(sid * rpt, rpt)],
                    acc_sh.at[pl.ds(sid * rpt, rpt)])
    pltpu.sync_copy(ones_hbm, ones_v)
    pltpu.sync_copy(dst_hbm.at[wid], idx_v)
    plsc.subcore_barrier()

    def body(c, carry):
      pltpu.sync_copy(ones_v, acc_sh.at[idx_v.at[c]], add=True)
      return carry

    lax.fori_loop(0, nch, body, 0)
    plsc.subcore_barrier()
    pltpu.sync_copy(acc_sh.at[pl.ds(sid * rpt, rpt)],
                    out_hbm.at[cid, pl.ds(sid * rpt, rpt)])

  return k(dst_w, ones_in, zeros)


def _sc_spmm(table, src_w, dst_w, zeros, npad):
  hsp, ntab, dh = table.shape
  nw, nch, ch = src_w.shape
  rpt = npad // NS
  rpt_tab = ntab // NS
  mesh = plsc.VectorSubcoreMesh(core_axis_name="c", subcore_axis_name="s", num_cores=NC, num_subcores=NS)

  @functools.partial(
      pl.kernel,
      out_type=jax.ShapeDtypeStruct((NC, hsp, npad, dh), jnp.float32),
      mesh=mesh,
      scratch_types=[
          pltpu.VMEM((nch, ch), jnp.int32),
          pltpu.VMEM((nch, ch), jnp.int32),
          pltpu.VMEM((NBUF, ch, dh), jnp.float32),
          pltpu.VMEM_SHARED((npad, dh), jnp.float32),
          pltpu.VMEM_SHARED((ntab, dh), jnp.float32),
          pltpu.SemaphoreType.DMA,
          pltpu.SemaphoreType.DMA,
      ],
      compiler_params=pltpu.CompilerParams(use_tc_tiling_on_sc=False),
  )
  def k(tab_hbm, src_hbm, dst_hbm, zero_hbm, out_hbm,
        sidx, didx, rows, acc_sh, tab_sh, sem0, sem1):
    sems = [sem0, sem1]
    cid = lax.axis_index("c")
    sid = lax.axis_index("s")
    wid = cid * NS + sid
    pltpu.sync_copy(src_hbm.at[wid], sidx)
    pltpu.sync_copy(dst_hbm.at[wid], didx)
    for h in range(hsp):
      pltpu.sync_copy(tab_hbm.at[h, pl.ds(sid * rpt_tab, rpt_tab)],
                      tab_sh.at[pl.ds(sid * rpt_tab, rpt_tab)])
      pltpu.sync_copy(zero_hbm.at[pl.ds(sid * rpt, rpt)],
                      acc_sh.at[pl.ds(sid * rpt, rpt)])
      plsc.subcore_barrier()
      tab_h = tab_sh
      for b in range(NBUF):
        pltpu.make_async_copy(tab_h.at[sidx.at[b]], rows.at[b],
                              sems[b]).start()

      def outer(i, carry):
        c0 = i * NBUF
        for b in range(NBUF):
          c = c0 + b
          pltpu.make_async_copy(tab_h.at[sidx.at[c]], rows.at[b],
                                sems[b]).wait()
          pltpu.sync_copy(rows.at[b], acc_sh.at[didx.at[c]], add=True)
          nxt = c + NBUF

          @pl.when(nxt < nch)
          def _():
            pltpu.make_async_copy(tab_h.at[sidx.at[nxt]], rows.at[b],
                                  sems[b]).start()

        return carry

      lax.fori_loop(0, nch // NBUF, outer, 0)
      plsc.subcore_barrier()
      pltpu.sync_copy(acc_sh.at[pl.ds(sid * rpt, rpt)],
                      out_hbm.at[cid, h, pl.ds(sid * rpt, rpt)])

  return k(table, src_w, dst_w, zeros)


def _tc_matmul(x, w):
  n, kdim = x.shape
  m = w.shape[1]
  bm = 2000

  def body(x_ref, w_ref, o_ref):
    o_ref[...] = jnp.dot(x_ref[...], w_ref[...],
                         preferred_element_type=jnp.float32)

  return pl.pallas_call(
      body,
      grid=(n // bm,),
      in_specs=[pl.BlockSpec((bm, kdim), lambda i: (i, 0)),
                pl.BlockSpec((kdim, m), lambda i: (0, 0))],
      out_specs=pl.BlockSpec((bm, m), lambda i: (i, 0)),
      out_shape=jax.ShapeDtypeStruct((n, m), jnp.float32),
  )(x, w)


def _tc_scale(deg, h1):
  _, n, _ = deg.shape
  hid = h1.shape[1]
  bm = 2000

  hsp = 4
  dh = hid // hsp

  def body(deg_ref, h_ref, dinv_ref, ht_ref):
    degsum = deg_ref[0] + deg_ref[1] + 1.0
    dinv16 = lax.rsqrt(jnp.maximum(degsum, 1e-12))
    dinv_b = jnp.broadcast_to(dinv16[:, :1], (bm, hid))
    dinv_ref[...] = dinv_b
    ht = dinv_b * h_ref[...]
    for h in range(hsp):
      ht_ref[h] = ht[:, h * dh:(h + 1) * dh]

  return pl.pallas_call(
      body,
      grid=(n // bm,),
      in_specs=[pl.BlockSpec((2, bm, 16), lambda i: (0, i, 0)),
                pl.BlockSpec((bm, hid), lambda i: (i, 0))],
      out_specs=[pl.BlockSpec((bm, hid), lambda i: (i, 0)),
                 pl.BlockSpec((hsp, bm, dh), lambda i: (0, i, 0))],
      out_shape=[jax.ShapeDtypeStruct((n, hid), jnp.float32),
                 jax.ShapeDtypeStruct((hsp, n, dh), jnp.float32)],
  )(deg, h1)


def _tc_layer2(acc1, ht1, dinv_b, b1, w2):
  _, hsp, n, dh = acc1.shape
  hid = hsp * dh
  ncls = w2.shape[1]
  bm = 2000

  def body(acc_ref, ht_ref, dinv_ref, b1_ref, w2_ref, o_ref):
    accsum = acc_ref[0] + acc_ref[1]
    acc = jnp.concatenate([accsum[h] for h in range(hsp)], axis=-1)
    ht = jnp.concatenate([ht_ref[h] for h in range(hsp)], axis=-1)
    agg = dinv_ref[...] * (acc + ht) + b1_ref[...]
    z = jnp.maximum(agg, 0.0)
    h2 = jnp.dot(z, w2_ref[...], preferred_element_type=jnp.float32)
    o_ref[0] = h2 * dinv_ref[:, :ncls]

  return pl.pallas_call(
      body,
      grid=(n // bm,),
      in_specs=[pl.BlockSpec((2, hsp, bm, dh), lambda i: (0, 0, i, 0)),
                pl.BlockSpec((hsp, bm, dh), lambda i: (0, i, 0)),
                pl.BlockSpec((bm, hid), lambda i: (i, 0)),
                pl.BlockSpec((1, hid), lambda i: (0, 0)),
                pl.BlockSpec((hid, ncls), lambda i: (0, 0))],
      out_specs=pl.BlockSpec((1, bm, ncls), lambda i: (0, i, 0)),
      out_shape=jax.ShapeDtypeStruct((1, n, ncls), jnp.float32),
  )(acc1, ht1, dinv_b, b1, w2)


def _tc_final(acc2, ht2, dinv_b, b2):
  _, _, n, ncls = acc2.shape
  hid = dinv_b.shape[1]
  bm = 2000

  def body(acc_ref, ht_ref, dinv_ref, b2_ref, o_ref):
    o = dinv_ref[:, :ncls] * (acc_ref[0, 0] + acc_ref[1, 0] + ht_ref[0])
    o = o + b2_ref[...]
    m = jnp.max(o, axis=-1, keepdims=True)
    ex = jnp.exp(o - m)
    lse = jnp.log(jnp.sum(ex, axis=-1, keepdims=True)) + m
    o_ref[...] = o - lse

  return pl.pallas_call(
      body,
      grid=(n // bm,),
      in_specs=[pl.BlockSpec((2, 1, bm, ncls), lambda i: (0, 0, i, 0)),
                pl.BlockSpec((1, bm, ncls), lambda i: (0, i, 0)),
                pl.BlockSpec((bm, hid), lambda i: (i, 0)),
                pl.BlockSpec((1, ncls), lambda i: (0, 0))],
      out_specs=pl.BlockSpec((bm, ncls), lambda i: (i, 0)),
      out_shape=jax.ShapeDtypeStruct((n, ncls), jnp.float32),
  )(acc2, ht2, dinv_b, b2)


def kernel(x, edge_index, W1, b1, W2, b2):
  n, _ = x.shape
  hid = W1.shape[1]
  ncls = W2.shape[1]
  e = edge_index.shape[1]
  nw = NC * NS
  npad = ((n + 1 + 127) // 128) * 128
  nch = -(-e // (nw * CHUNK))
  nch = ((nch + NBUF - 1) // NBUF) * NBUF
  epad = nw * nch * CHUNK

  src = edge_index[0]
  dst = edge_index[1]
  pad = epad - e
  srcp = jnp.concatenate([src, jnp.zeros((pad,), jnp.int32)]).reshape(
      nw, nch, CHUNK)
  dstp = jnp.concatenate([dst, jnp.full((pad,), n, jnp.int32)]).reshape(
      nw, nch, CHUNK)
  zeros16 = jnp.zeros((npad, 16), jnp.float32)
  zeros_h = jnp.zeros((npad, hid // 4), jnp.float32)
  zeros_c = jnp.zeros((npad, ncls), jnp.float32)
  ones_in = jnp.ones((CHUNK, 16), jnp.float32)

  deg_parts = _sc_degree(dstp, ones_in, zeros16, npad)
  h1 = _tc_matmul(x, W1)
  dinv_b, ht1 = _tc_scale(deg_parts[:, :n], h1)
  acc1 = _sc_spmm(ht1, srcp, dstp, zeros_h, npad)
  ht2 = _tc_layer2(acc1[:, :, :n], ht1, dinv_b, b1.reshape(1, -1), W2)
  acc2 = _sc_spmm(ht2, srcp, dstp, zeros_c, npad)
  return _tc_final(acc2[:, :, :n], ht2, dinv_b, b2.reshape(1, -1))


# speedup vs baseline: 2.3721x; 1.2432x over previous
import functools

import jax
import jax.numpy as jnp
from jax import lax
from jax.experimental import pallas as pl
from jax.experimental.pallas import tpu as pltpu
from jax.experimental.pallas import tpu_sc as plsc

NC = 2
NS = 16
CHUNK = 1000
NBUF = 2


def _sc_degree(dst_w, ones_in, zeros, npad):
  nw, nch, ch = dst_w.shape
  rpt = npad // NS
  mesh = plsc.VectorSubcoreMesh(core_axis_name="c", subcore_axis_name="s", num_cores=NC, num_subcores=NS)

  @functools.partial(
      pl.kernel,
      out_type=jax.ShapeDtypeStruct((NC, npad, 16), jnp.float32),
      mesh=mesh,
      scratch_types=[
          pltpu.VMEM((nch, ch), jnp.int32),
          pltpu.VMEM((ch, 16), jnp.float32),
          pltpu.VMEM_SHARED((npad, 16), jnp.float32),
      ],
      compiler_params=pltpu.CompilerParams(use_tc_tiling_on_sc=False),
  )
  def k(dst_hbm, ones_hbm, zero_hbm, out_hbm, idx_v, ones_v, acc_sh):
    cid = lax.axis_index("c")
    sid = lax.axis_index("s")
    wid = cid * NS + sid
    pltpu.sync_copy(zero_hbm.at[pl.ds(sid * rpt, rpt)],
                    acc_sh.at[pl.ds(sid * rpt, rpt)])
    pltpu.sync_copy(ones_hbm, ones_v)
    pltpu.sync_copy(dst_hbm.at[wid], idx_v)
    plsc.subcore_barrier()

    def body(c, carry):
      pltpu.sync_copy(ones_v, acc_sh.at[idx_v.at[c]], add=True)
      return carry

    lax.fori_loop(0, nch, body, 0)
    plsc.subcore_barrier()
    pltpu.sync_copy(acc_sh.at[pl.ds(sid * rpt, rpt)],
                    out_hbm.at[cid, pl.ds(sid * rpt, rpt)])

  return k(dst_w, ones_in, zeros)


def _sc_spmm(table, src_w, dst_w, zeros, npad):
  hsp, ntab, dh = table.shape
  nw, nch, ch = src_w.shape
  rpt = npad // NS
  rpt_tab = ntab // NS
  mesh = plsc.VectorSubcoreMesh(core_axis_name="c", subcore_axis_name="s", num_cores=NC, num_subcores=NS)

  @functools.partial(
      pl.kernel,
      out_type=jax.ShapeDtypeStruct((NC, hsp, npad, dh), jnp.float32),
      mesh=mesh,
      scratch_types=[
          pltpu.VMEM((nch, ch), jnp.int32),
          pltpu.VMEM((nch, ch), jnp.int32),
          pltpu.VMEM((NBUF, ch, dh), jnp.float32),
          pltpu.VMEM_SHARED((npad, dh), jnp.float32),
          pltpu.VMEM_SHARED((ntab, dh), jnp.float32),
          pltpu.SemaphoreType.DMA,
          pltpu.SemaphoreType.DMA,
      ],
      compiler_params=pltpu.CompilerParams(use_tc_tiling_on_sc=False),
  )
  def k(tab_hbm, src_hbm, dst_hbm, zero_hbm, out_hbm,
        sidx, didx, rows, acc_sh, tab_sh, sem0, sem1):
    sems = [sem0, sem1]
    cid = lax.axis_index("c")
    sid = lax.axis_index("s")
    wid = cid * NS + sid
    pltpu.sync_copy(src_hbm.at[wid], sidx)
    pltpu.sync_copy(dst_hbm.at[wid], didx)
    for h in range(hsp):
      pltpu.sync_copy(tab_hbm.at[h, pl.ds(sid * rpt_tab, rpt_tab)],
                      tab_sh.at[pl.ds(sid * rpt_tab, rpt_tab)])
      pltpu.sync_copy(zero_hbm.at[pl.ds(sid * rpt, rpt)],
                      acc_sh.at[pl.ds(sid * rpt, rpt)])
      plsc.subcore_barrier()
      tab_h = tab_sh
      for b in range(NBUF):
        pltpu.make_async_copy(tab_h.at[sidx.at[b]], rows.at[b],
                              sems[b]).start()

      def outer(i, carry):
        c0 = i * NBUF
        for b in range(NBUF):
          c = c0 + b
          pltpu.make_async_copy(tab_h.at[sidx.at[c]], rows.at[b],
                                sems[b]).wait()
          pltpu.sync_copy(rows.at[b], acc_sh.at[didx.at[c]], add=True)
          nxt = c + NBUF

          @pl.when(nxt < nch)
          def _():
            pltpu.make_async_copy(tab_h.at[sidx.at[nxt]], rows.at[b],
                                  sems[b]).start()

        return carry

      lax.fori_loop(0, nch // NBUF, outer, 0)
      plsc.subcore_barrier()
      pltpu.sync_copy(acc_sh.at[pl.ds(sid * rpt, rpt)],
                      out_hbm.at[cid, h, pl.ds(sid * rpt, rpt)])

  return k(table, src_w, dst_w, zeros)


def _tc_matmul(x, w):
  n, kdim = x.shape
  m = w.shape[1]
  bm = 2000

  def body(x_ref, w_ref, o_ref):
    o_ref[...] = jnp.dot(x_ref[...], w_ref[...],
                         preferred_element_type=jnp.float32)

  return pl.pallas_call(
      body,
      grid=(n // bm,),
      in_specs=[pl.BlockSpec((bm, kdim), lambda i: (i, 0)),
                pl.BlockSpec((kdim, m), lambda i: (0, 0))],
      out_specs=pl.BlockSpec((bm, m), lambda i: (i, 0)),
      out_shape=jax.ShapeDtypeStruct((n, m), jnp.float32),
  )(x, w)


def _tc_scale(deg, h1):
  hid = h1.shape[1]
  n = h1.shape[0]
  bm = 2000

  hsp = 4
  dh = hid // hsp

  def body(deg_ref, h_ref, dinv_ref, ht_ref):
    degsum = deg_ref[0] + deg_ref[1] + 1.0
    dinv16 = lax.rsqrt(jnp.maximum(degsum, 1e-12))
    dinv_ref[...] = dinv16
    ht = jnp.broadcast_to(dinv16[:, :1], (bm, hid)) * h_ref[...]
    for h in range(hsp):
      ht_ref[h] = ht[:, h * dh:(h + 1) * dh]

  return pl.pallas_call(
      body,
      grid=(n // bm,),
      in_specs=[pl.BlockSpec((2, bm, 16), lambda i: (0, i, 0)),
                pl.BlockSpec((bm, hid), lambda i: (i, 0))],
      out_specs=[pl.BlockSpec((bm, 16), lambda i: (i, 0)),
                 pl.BlockSpec((hsp, bm, dh), lambda i: (0, i, 0))],
      out_shape=[jax.ShapeDtypeStruct((n, 16), jnp.float32),
                 jax.ShapeDtypeStruct((hsp, n, dh), jnp.float32)],
  )(deg, h1)


def _tc_layer2(acc1, ht1, dinv16, b1, w2):
  _, hsp, _, dh = acc1.shape
  hid = hsp * dh
  n = dinv16.shape[0]
  ncls = w2.shape[1]
  bm = 2000

  def body(acc_ref, ht_ref, dinv_ref, b1_ref, w2_ref, o_ref):
    accsum = acc_ref[0] + acc_ref[1]
    acc = jnp.concatenate([accsum[h] for h in range(hsp)], axis=-1)
    ht = jnp.concatenate([ht_ref[h] for h in range(hsp)], axis=-1)
    dinv_b = jnp.broadcast_to(dinv_ref[:, :1], (bm, hid))
    agg = dinv_b * (acc + ht) + b1_ref[...]
    z = jnp.maximum(agg, 0.0)
    h2 = jnp.dot(z, w2_ref[...], preferred_element_type=jnp.float32)
    o_ref[0] = h2 * dinv_ref[:, :ncls]

  return pl.pallas_call(
      body,
      grid=(n // bm,),
      in_specs=[pl.BlockSpec((2, hsp, bm, dh), lambda i: (0, 0, i, 0)),
                pl.BlockSpec((hsp, bm, dh), lambda i: (0, i, 0)),
                pl.BlockSpec((bm, 16), lambda i: (i, 0)),
                pl.BlockSpec((1, hid), lambda i: (0, 0)),
                pl.BlockSpec((hid, ncls), lambda i: (0, 0))],
      out_specs=pl.BlockSpec((1, bm, ncls), lambda i: (0, i, 0)),
      out_shape=jax.ShapeDtypeStruct((1, n, ncls), jnp.float32),
  )(acc1, ht1, dinv16, b1, w2)


def _tc_final(acc2, ht2, dinv16, b2):
  ncls = acc2.shape[3]
  n = dinv16.shape[0]
  bm = 2000

  def body(acc_ref, ht_ref, dinv_ref, b2_ref, o_ref):
    o = dinv_ref[:, :ncls] * (acc_ref[0, 0] + acc_ref[1, 0] + ht_ref[0])
    o = o + b2_ref[...]
    m = jnp.max(o, axis=-1, keepdims=True)
    ex = jnp.exp(o - m)
    lse = jnp.log(jnp.sum(ex, axis=-1, keepdims=True)) + m
    o_ref[...] = o - lse

  return pl.pallas_call(
      body,
      grid=(n // bm,),
      in_specs=[pl.BlockSpec((2, 1, bm, ncls), lambda i: (0, 0, i, 0)),
                pl.BlockSpec((1, bm, ncls), lambda i: (0, i, 0)),
                pl.BlockSpec((bm, 16), lambda i: (i, 0)),
                pl.BlockSpec((1, ncls), lambda i: (0, 0))],
      out_specs=pl.BlockSpec((bm, ncls), lambda i: (i, 0)),
      out_shape=jax.ShapeDtypeStruct((n, ncls), jnp.float32),
  )(acc2, ht2, dinv16, b2)


def kernel(x, edge_index, W1, b1, W2, b2):
  n, _ = x.shape
  hid = W1.shape[1]
  ncls = W2.shape[1]
  e = edge_index.shape[1]
  nw = NC * NS
  npad = ((n + 1 + 127) // 128) * 128
  nch = -(-e // (nw * CHUNK))
  nch = ((nch + NBUF - 1) // NBUF) * NBUF
  epad = nw * nch * CHUNK

  src = edge_index[0]
  dst = edge_index[1]
  pad = epad - e
  if pad:
    src = jnp.concatenate([src, jnp.zeros((pad,), jnp.int32)])
    dst = jnp.concatenate([dst, jnp.full((pad,), n, jnp.int32)])
  srcp = src.reshape(nw, nch, CHUNK)
  dstp = dst.reshape(nw, nch, CHUNK)
  zeros16 = jnp.zeros((npad, 16), jnp.float32)
  zeros_h = jnp.zeros((npad, hid // 4), jnp.float32)
  zeros_c = jnp.zeros((npad, ncls), jnp.float32)
  ones_in = jnp.ones((CHUNK, 16), jnp.float32)

  deg_parts = _sc_degree(dstp, ones_in, zeros16, npad)
  h1 = _tc_matmul(x, W1)
  dinv16, ht1 = _tc_scale(deg_parts, h1)
  acc1 = _sc_spmm(ht1, srcp, dstp, zeros_h, npad)
  ht2 = _tc_layer2(acc1, ht1, dinv16, b1.reshape(1, -1), W2)
  acc2 = _sc_spmm(ht2, srcp, dstp, zeros_c, npad)
  return _tc_final(acc2, ht2, dinv16, b2.reshape(1, -1))


# speedup vs baseline: 2.5207x; 1.0626x over previous
import functools

import jax
import jax.numpy as jnp
from jax import lax
from jax.experimental import pallas as pl
from jax.experimental.pallas import tpu as pltpu
from jax.experimental.pallas import tpu_sc as plsc

NC = 2
NS = 16
CHUNK = 128
NBUF = 2


def _sc_degree(dst_w, ones_in, zeros, npad):
  nw, nch, ch = dst_w.shape
  rpt = npad // NS
  mesh = plsc.VectorSubcoreMesh(core_axis_name="c", subcore_axis_name="s", num_cores=NC, num_subcores=NS)

  @functools.partial(
      pl.kernel,
      out_type=jax.ShapeDtypeStruct((NC, npad, 16), jnp.float32),
      mesh=mesh,
      scratch_types=[
          pltpu.VMEM((nch, ch), jnp.int32),
          pltpu.VMEM((ch, 16), jnp.float32),
          pltpu.VMEM_SHARED((npad, 16), jnp.float32),
      ],
      compiler_params=pltpu.CompilerParams(use_tc_tiling_on_sc=False),
  )
  def k(dst_hbm, ones_hbm, zero_hbm, out_hbm, idx_v, ones_v, acc_sh):
    cid = lax.axis_index("c")
    sid = lax.axis_index("s")
    wid = cid * NS + sid
    pltpu.sync_copy(zero_hbm.at[pl.ds(sid * rpt, rpt)],
                    acc_sh.at[pl.ds(sid * rpt, rpt)])
    pltpu.sync_copy(ones_hbm, ones_v)
    pltpu.sync_copy(dst_hbm.at[wid], idx_v)
    plsc.subcore_barrier()

    def body(c, carry):
      pltpu.sync_copy(ones_v, acc_sh.at[idx_v.at[c]], add=True)
      return carry

    lax.fori_loop(0, nch, body, 0)
    plsc.subcore_barrier()
    pltpu.sync_copy(acc_sh.at[pl.ds(sid * rpt, rpt)],
                    out_hbm.at[cid, pl.ds(sid * rpt, rpt)])

  return k(dst_w, ones_in, zeros)


def _sc_spmm(table, src_w, dst_w, zeros, npad, dh):
  ntab, d = table.shape
  hsp = d // dh
  nw, nch, ch = src_w.shape
  rpt = npad // NS
  rpt_tab = ntab // NS
  mesh = plsc.VectorSubcoreMesh(core_axis_name="c", subcore_axis_name="s", num_cores=NC, num_subcores=NS)

  @functools.partial(
      pl.kernel,
      out_type=jax.ShapeDtypeStruct((NC, npad, d), jnp.float32),
      mesh=mesh,
      scratch_types=[
          pltpu.VMEM((nch, ch), jnp.int32),
          pltpu.VMEM((nch, ch), jnp.int32),
          pltpu.VMEM((NBUF, ch, dh), jnp.float32),
          pltpu.VMEM_SHARED((npad, dh), jnp.float32),
          pltpu.VMEM_SHARED((ntab, dh), jnp.float32),
          pltpu.SemaphoreType.DMA,
          pltpu.SemaphoreType.DMA,
      ],
      compiler_params=pltpu.CompilerParams(use_tc_tiling_on_sc=False),
  )
  def k(tab_hbm, src_hbm, dst_hbm, zero_hbm, out_hbm,
        sidx, didx, rows, acc_sh, tab_sh, sem0, sem1):
    sems = [sem0, sem1]
    cid = lax.axis_index("c")
    sid = lax.axis_index("s")
    wid = cid * NS + sid
    pltpu.sync_copy(src_hbm.at[wid], sidx)
    pltpu.sync_copy(dst_hbm.at[wid], didx)
    for h in range(hsp):
      pltpu.sync_copy(
          tab_hbm.at[pl.ds(sid * rpt_tab, rpt_tab), pl.ds(h * dh, dh)],
          tab_sh.at[pl.ds(sid * rpt_tab, rpt_tab)])
      pltpu.sync_copy(zero_hbm.at[pl.ds(sid * rpt, rpt)],
                      acc_sh.at[pl.ds(sid * rpt, rpt)])
      plsc.subcore_barrier()
      for b in range(NBUF):
        pltpu.make_async_copy(tab_sh.at[sidx.at[b]], rows.at[b],
                              sems[b]).start()

      def outer(i, carry):
        c0 = i * NBUF
        for b in range(NBUF):
          c = c0 + b
          pltpu.make_async_copy(tab_sh.at[sidx.at[c]], rows.at[b],
                                sems[b]).wait()
          pltpu.sync_copy(rows.at[b], acc_sh.at[didx.at[c]], add=True)
          nxt = c + NBUF

          @pl.when(nxt < nch)
          def _():
            pltpu.make_async_copy(tab_sh.at[sidx.at[nxt]], rows.at[b],
                                  sems[b]).start()

        return carry

      lax.fori_loop(0, nch // NBUF, outer, 0)
      plsc.subcore_barrier()
      pltpu.sync_copy(acc_sh.at[pl.ds(sid * rpt, rpt)],
                      out_hbm.at[cid, pl.ds(sid * rpt, rpt),
                                 pl.ds(h * dh, dh)])

  return k(table, src_w, dst_w, zeros)


def _tc_matmul(x, w):
  n, kdim = x.shape
  m = w.shape[1]
  bm = 2000

  def body(x_ref, w_ref, o_ref):
    o_ref[...] = jnp.dot(x_ref[...], w_ref[...],
                         preferred_element_type=jnp.float32)

  return pl.pallas_call(
      body,
      grid=(n // bm,),
      in_specs=[pl.BlockSpec((bm, kdim), lambda i: (i, 0)),
                pl.BlockSpec((kdim, m), lambda i: (0, 0))],
      out_specs=pl.BlockSpec((bm, m), lambda i: (i, 0)),
      out_shape=jax.ShapeDtypeStruct((n, m), jnp.float32),
  )(x, w)


def _tc_scale(deg, h1):
  hid = h1.shape[1]
  n = h1.shape[0]
  bm = 2000

  def body(deg_ref, h_ref, dinv_ref, ht_ref):
    degsum = deg_ref[0] + deg_ref[1] + 1.0
    dinv16 = lax.rsqrt(jnp.maximum(degsum, 1e-12))
    dinv_ref[...] = dinv16
    ht_ref[...] = jnp.broadcast_to(dinv16[:, :1], (bm, hid)) * h_ref[...]

  return pl.pallas_call(
      body,
      grid=(n // bm,),
      in_specs=[pl.BlockSpec((2, bm, 16), lambda i: (0, i, 0)),
                pl.BlockSpec((bm, hid), lambda i: (i, 0))],
      out_specs=[pl.BlockSpec((bm, 16), lambda i: (i, 0)),
                 pl.BlockSpec((bm, hid), lambda i: (i, 0))],
      out_shape=[jax.ShapeDtypeStruct((n, 16), jnp.float32),
                 jax.ShapeDtypeStruct((n, hid), jnp.float32)],
  )(deg, h1)


def _tc_layer2(acc1, ht1, dinv16, b1, w2):
  hid = acc1.shape[2]
  n = dinv16.shape[0]
  ncls = w2.shape[1]
  bm = 2000

  def body(acc_ref, ht_ref, dinv_ref, b1_ref, w2_ref, o_ref):
    acc = acc_ref[0] + acc_ref[1]
    dinv_b = jnp.broadcast_to(dinv_ref[:, :1], (bm, hid))
    agg = dinv_b * (acc + ht_ref[...]) + b1_ref[...]
    z = jnp.maximum(agg, 0.0)
    h2 = jnp.dot(z, w2_ref[...], preferred_element_type=jnp.float32)
    o_ref[...] = h2 * dinv_ref[:, :ncls]

  return pl.pallas_call(
      body,
      grid=(n // bm,),
      in_specs=[pl.BlockSpec((2, bm, hid), lambda i: (0, i, 0)),
                pl.BlockSpec((bm, hid), lambda i: (i, 0)),
                pl.BlockSpec((bm, 16), lambda i: (i, 0)),
                pl.BlockSpec((1, hid), lambda i: (0, 0)),
                pl.BlockSpec((hid, ncls), lambda i: (0, 0))],
      out_specs=pl.BlockSpec((bm, ncls), lambda i: (i, 0)),
      out_shape=jax.ShapeDtypeStruct((n, ncls), jnp.float32),
  )(acc1, ht1, dinv16, b1, w2)


def _tc_final(acc2, ht2, dinv16, b2):
  ncls = acc2.shape[2]
  n = dinv16.shape[0]
  bm = 2000

  def body(acc_ref, ht_ref, dinv_ref, b2_ref, o_ref):
    o = dinv_ref[:, :ncls] * (acc_ref[0] + acc_ref[1] + ht_ref[...])
    o = o + b2_ref[...]
    m = jnp.max(o, axis=-1, keepdims=True)
    ex = jnp.exp(o - m)
    lse = jnp.log(jnp.sum(ex, axis=-1, keepdims=True)) + m
    o_ref[...] = o - lse

  return pl.pallas_call(
      body,
      grid=(n // bm,),
      in_specs=[pl.BlockSpec((2, bm, ncls), lambda i: (0, i, 0)),
                pl.BlockSpec((bm, ncls), lambda i: (i, 0)),
                pl.BlockSpec((bm, 16), lambda i: (i, 0)),
                pl.BlockSpec((1, ncls), lambda i: (0, 0))],
      out_specs=pl.BlockSpec((bm, ncls), lambda i: (i, 0)),
      out_shape=jax.ShapeDtypeStruct((n, ncls), jnp.float32),
  )(acc2, ht2, dinv16, b2)


def kernel(x, edge_index, W1, b1, W2, b2):
  n, _ = x.shape
  hid = W1.shape[1]
  ncls = W2.shape[1]
  e = edge_index.shape[1]
  nw = NC * NS
  npad = ((n + 1 + 127) // 128) * 128
  nch = -(-e // (nw * CHUNK))
  nch = ((nch + NBUF - 1) // NBUF) * NBUF
  epad = nw * nch * CHUNK

  src = edge_index[0]
  dst = edge_index[1]
  pad = epad - e
  if pad:
    src = jnp.concatenate([src, jnp.zeros((pad,), jnp.int32)])
    dst = jnp.concatenate([dst, jnp.full((pad,), n, jnp.int32)])
  srcp = src.reshape(nw, nch, CHUNK)
  dstp = dst.reshape(nw, nch, CHUNK)
  zeros16 = jnp.zeros((npad, 16), jnp.float32)
  zeros_h = jnp.zeros((npad, hid // 4), jnp.float32)
  zeros_c = jnp.zeros((npad, ncls), jnp.float32)
  ones_in = jnp.ones((CHUNK, 16), jnp.float32)

  deg_parts = _sc_degree(dstp, ones_in, zeros16, npad)
  h1 = _tc_matmul(x, W1)
  dinv16, ht1 = _tc_scale(deg_parts, h1)
  acc1 = _sc_spmm(ht1, srcp, dstp, zeros_h, npad, hid // 4)
  ht2 = _tc_layer2(acc1, ht1, dinv16, b1.reshape(1, -1), W2)
  acc2 = _sc_spmm(ht2, srcp, dstp, zeros_c, npad, ncls)
  return _tc_final(acc2, ht2, dinv16, b2.reshape(1, -1))
